# Initial kernel scaffold; baseline (speedup 1.0000x reference)
import jax
import jax.numpy as jnp
from jax.experimental import pallas as pl


def kernel(node_feats, edge_index, W, b):
    raise NotImplementedError("write your pallas kernel here")


# speedup vs baseline: 5.5418x; 5.5418x over previous
import functools

import jax
import jax.numpy as jnp
from jax import lax
from jax.experimental import pallas as pl
from jax.experimental.pallas import tpu as pltpu
from jax.experimental.pallas import tpu_sc as plsc

N = 10000
D = 128
E = 320000
EPS = 1e-07

NC = 2
NS = 16
CHUNK = 128
NCH = 160
EPT = NCH * CHUNK
EPAD = NS * EPT
NPAD = 10240
IGRP = 32
RPT = NPAD // NS
ZROWS = 128


def _prologue_body(x_ref, table_ref):
    x = x_ref[...]
    g = jnp.maximum(x, 0.0) + EPS
    gmax = jnp.max(g, axis=0, keepdims=True)
    eg = jnp.exp(g - gmax)
    table_ref[0:N, :] = eg
    table_ref[N:2 * N, :] = g * eg
    table_ref[2 * N:, :] = jnp.zeros((8, D), jnp.float32)


_prologue = pl.pallas_call(
    _prologue_body,
    out_shape=jax.ShapeDtypeStruct((2 * N + 8, D), jnp.float32),
)


_sc_mesh = plsc.VectorSubcoreMesh(core_axis_name="c", subcore_axis_name="s")


@functools.partial(
    pl.kernel,
    mesh=_sc_mesh,
    out_type=jax.ShapeDtypeStruct((NC, NPAD, D), jnp.float32),
    scratch_types=[
        pltpu.VMEM((IGRP, CHUNK), jnp.int32),
        pltpu.VMEM((IGRP, CHUNK), jnp.int32),
        pltpu.VMEM((CHUNK, D), jnp.float32),
        pltpu.VMEM_SHARED((NPAD, D), jnp.float32),
        pltpu.SemaphoreType.DMA,
    ],
)
def _edge_kernel(table_hbm, gidx_hbm, sidx_hbm, out_hbm,
                 gidx_v, sidx_v, rows_v, acc_sh, sem):
    c = lax.axis_index("c")
    s = lax.axis_index("s")
    base = s * RPT

    def _zero_body(i, carry):
        for j in range(D // 16):
            rows_v[i, pl.ds(j * 16, 16)] = jnp.zeros((16,), jnp.float32)
        return carry

    lax.fori_loop(0, CHUNK, _zero_body, 0)
    for k in range(RPT // ZROWS):
        pltpu.sync_copy(rows_v,
                        acc_sh.at[pl.ds(base + k * ZROWS, ZROWS)])
    plsc.subcore_barrier()

    def _chunk_body(j, carry):
        pltpu.async_copy(table_hbm.at[gidx_v.at[j]], rows_v, sem).wait()
        pltpu.sync_copy(rows_v, acc_sh.at[sidx_v.at[j]], add=True)
        return carry

    for grp in range(NCH // IGRP):
        pltpu.sync_copy(gidx_hbm.at[c, s, pl.ds(grp * IGRP, IGRP)], gidx_v)
        pltpu.sync_copy(sidx_hbm.at[s, pl.ds(grp * IGRP, IGRP)], sidx_v)
        lax.fori_loop(0, IGRP, _chunk_body, 0)
    plsc.subcore_barrier()

    pltpu.sync_copy(acc_sh.at[pl.ds(base, RPT)],
                    out_hbm.at[c, pl.ds(base, RPT)])


def _epilogue_body(acc_ref, x_ref, w_ref, b_ref, o_ref):
    sarr = acc_ref[0]
    tarr = acc_ref[1]
    x = x_ref[...]
    agg = tarr / (sarr + 1e-16)
    an = jnp.sqrt(jnp.sum(agg * agg, axis=1, keepdims=True))
    msg = agg / jnp.maximum(an, 1e-12)
    xn = jnp.sqrt(jnp.sum(x * x, axis=1, keepdims=True))
    feats = x + msg * xn
    o_ref[...] = lax.dot_general(
        feats, w_ref[...], (((1,), (1,)), ((), ())),
        preferred_element_type=jnp.float32) + b_ref[...]


_RB = 1000

_epilogue = pl.pallas_call(
    _epilogue_body,
    grid=(N // _RB,),
    in_specs=[
        pl.BlockSpec((NC, _RB, D), lambda i: (0, i, 0)),
        pl.BlockSpec((_RB, D), lambda i: (i, 0)),
        pl.BlockSpec((D, D), lambda i: (0, 0)),
        pl.BlockSpec((1, D), lambda i: (0, 0)),
    ],
    out_specs=pl.BlockSpec((_RB, D), lambda i: (i, 0)),
    out_shape=jax.ShapeDtypeStruct((N, D), jnp.float32),
)


def kernel(node_feats, edge_index, W, b):
    src = edge_index[0]
    dst = edge_index[1]

    padg = jnp.full((EPAD - E,), 2 * N, jnp.int32)
    gidx = jnp.stack([
        jnp.concatenate([src, padg]),
        jnp.concatenate([src + N, padg]),
    ]).reshape(NC, NS, NCH, CHUNK)
    sidx = jnp.concatenate(
        [dst, jnp.zeros((EPAD - E,), jnp.int32)]).reshape(NS, NCH, CHUNK)

    table = _prologue(node_feats)
    acc = _edge_kernel(table, gidx, sidx)
    return _epilogue(acc, node_feats, W, b.reshape(1, D))


# speedup vs baseline: 6.0243x; 1.0871x over previous
import functools

import jax
import jax.numpy as jnp
from jax import lax
from jax.experimental import pallas as pl
from jax.experimental.pallas import tpu as pltpu
from jax.experimental.pallas import tpu_sc as plsc

N = 10000
D = 128
E = 320000
EPS = 1e-07

NC = 2
NS = 16
CHUNK = 128
NCH = 160
EPT = NCH * CHUNK
EPAD = NS * EPT
NPAD = 10240
IGRP = 16
RPT = NPAD // NS
ZROWS = 128


def _prologue_body(x_ref, table_ref):
    x = x_ref[...]
    g = jnp.maximum(x, 0.0) + EPS
    gmax = jnp.max(g, axis=0, keepdims=True)
    eg = jnp.exp(g - gmax)
    table_ref[0:N, :] = eg
    table_ref[N:2 * N, :] = g * eg
    table_ref[2 * N:, :] = jnp.zeros((8, D), jnp.float32)


_prologue = pl.pallas_call(
    _prologue_body,
    out_shape=jax.ShapeDtypeStruct((2 * N + 8, D), jnp.float32),
)


_sc_mesh = plsc.VectorSubcoreMesh(core_axis_name="c", subcore_axis_name="s")


@functools.partial(
    pl.kernel,
    mesh=_sc_mesh,
    out_type=jax.ShapeDtypeStruct((NC, NPAD, D), jnp.float32),
    scratch_types=[
        pltpu.VMEM((IGRP, CHUNK), jnp.int32),
        pltpu.VMEM((IGRP, CHUNK), jnp.int32),
        pltpu.VMEM((CHUNK, D), jnp.float32),
        pltpu.VMEM((CHUNK, D), jnp.float32),
        pltpu.VMEM_SHARED((NPAD, D), jnp.float32),
        pltpu.SemaphoreType.DMA,
        pltpu.SemaphoreType.DMA,
    ],
)
def _edge_kernel(table_hbm, gidx_hbm, sidx_hbm, out_hbm,
                 gidx_v, sidx_v, rows0_v, rows1_v, acc_sh, sem0, sem1):
    c = lax.axis_index("c")
    s = lax.axis_index("s")
    base = s * RPT

    def _zero_body(i, carry):
        for j in range(D // 16):
            rows0_v[i, pl.ds(j * 16, 16)] = jnp.zeros((16,), jnp.float32)
        return carry

    lax.fori_loop(0, CHUNK, _zero_body, 0)
    for k in range(RPT // ZROWS):
        pltpu.sync_copy(rows0_v,
                        acc_sh.at[pl.ds(base + k * ZROWS, ZROWS)])
    plsc.subcore_barrier()

    npairs = IGRP // 2

    def _pair_body(k, carry):
        pltpu.async_copy(table_hbm.at[gidx_v.at[2 * k + 1]], rows1_v, sem1)
        pltpu.make_async_copy(table_hbm.at[gidx_v.at[2 * k]],
                              rows0_v, sem0).wait()
        pltpu.sync_copy(rows0_v, acc_sh.at[sidx_v.at[2 * k]], add=True)

        @pl.when(k + 1 < npairs)
        def _():
            pltpu.async_copy(table_hbm.at[gidx_v.at[2 * k + 2]],
                             rows0_v, sem0)

        pltpu.make_async_copy(table_hbm.at[gidx_v.at[2 * k + 1]],
                              rows1_v, sem1).wait()
        pltpu.sync_copy(rows1_v, acc_sh.at[sidx_v.at[2 * k + 1]], add=True)
        return carry

    for grp in range(NCH // IGRP):
        pltpu.sync_copy(gidx_hbm.at[c, s, pl.ds(grp * IGRP, IGRP)], gidx_v)
        pltpu.sync_copy(sidx_hbm.at[s, pl.ds(grp * IGRP, IGRP)], sidx_v)
        pltpu.async_copy(table_hbm.at[gidx_v.at[0]], rows0_v, sem0)
        lax.fori_loop(0, npairs, _pair_body, 0)
    plsc.subcore_barrier()

    pltpu.sync_copy(acc_sh.at[pl.ds(base, RPT)],
                    out_hbm.at[c, pl.ds(base, RPT)])


def _epilogue_body(acc_ref, x_ref, w_ref, b_ref, o_ref):
    sarr = acc_ref[0]
    tarr = acc_ref[1]
    x = x_ref[...]
    agg = tarr / (sarr + 1e-16)
    an = jnp.sqrt(jnp.sum(agg * agg, axis=1, keepdims=True))
    msg = agg / jnp.maximum(an, 1e-12)
    xn = jnp.sqrt(jnp.sum(x * x, axis=1, keepdims=True))
    feats = x + msg * xn
    o_ref[...] = lax.dot_general(
        feats, w_ref[...], (((1,), (1,)), ((), ())),
        preferred_element_type=jnp.float32) + b_ref[...]


_RB = 1000

_epilogue = pl.pallas_call(
    _epilogue_body,
    grid=(N // _RB,),
    in_specs=[
        pl.BlockSpec((NC, _RB, D), lambda i: (0, i, 0)),
        pl.BlockSpec((_RB, D), lambda i: (i, 0)),
        pl.BlockSpec((D, D), lambda i: (0, 0)),
        pl.BlockSpec((1, D), lambda i: (0, 0)),
    ],
    out_specs=pl.BlockSpec((_RB, D), lambda i: (i, 0)),
    out_shape=jax.ShapeDtypeStruct((N, D), jnp.float32),
)


def kernel(node_feats, edge_index, W, b):
    src = edge_index[0]
    dst = edge_index[1]

    padg = jnp.full((EPAD - E,), 2 * N, jnp.int32)
    gidx = jnp.stack([
        jnp.concatenate([src, padg]),
        jnp.concatenate([src + N, padg]),
    ]).reshape(NC, NS, NCH, CHUNK)
    sidx = jnp.concatenate(
        [dst, jnp.zeros((EPAD - E,), jnp.int32)]).reshape(NS, NCH, CHUNK)

    table = _prologue(node_feats)
    acc = _edge_kernel(table, gidx, sidx)
    return _epilogue(acc, node_feats, W, b.reshape(1, D))


# speedup vs baseline: 6.0274x; 1.0005x over previous
import functools

import jax
import jax.numpy as jnp
from jax import lax
from jax.experimental import pallas as pl
from jax.experimental.pallas import tpu as pltpu
from jax.experimental.pallas import tpu_sc as plsc

N = 10000
D = 128
E = 320000
EPS = 1e-07

NC = 2
NS = 16
CHUNK = 64
NBUF = 4
NCH = 320
IGRP = 32
EPT = NCH * CHUNK
EPAD = NS * EPT
NPAD = 10240
RPT = NPAD // NS
ZROWS = 64


def _prologue_body(x_ref, table_ref):
    x = x_ref[...]
    g = jnp.maximum(x, 0.0) + EPS
    gmax = jnp.max(g, axis=0, keepdims=True)
    eg = jnp.exp(g - gmax)
    table_ref[0:N, :] = eg
    table_ref[N:2 * N, :] = g * eg
    table_ref[2 * N:, :] = jnp.zeros((8, D), jnp.float32)


_prologue = pl.pallas_call(
    _prologue_body,
    out_shape=jax.ShapeDtypeStruct((2 * N + 8, D), jnp.float32),
)


_sc_mesh = plsc.VectorSubcoreMesh(core_axis_name="c", subcore_axis_name="s")


@functools.partial(
    pl.kernel,
    mesh=_sc_mesh,
    out_type=jax.ShapeDtypeStruct((NC, NPAD, D), jnp.float32),
    scratch_types=[
        pltpu.VMEM((IGRP, CHUNK), jnp.int32),
        pltpu.VMEM((IGRP, CHUNK), jnp.int32),
        pltpu.VMEM((NBUF, CHUNK, D), jnp.float32),
        pltpu.VMEM_SHARED((NPAD, D), jnp.float32),
        pltpu.SemaphoreType.DMA,
        pltpu.SemaphoreType.DMA,
        pltpu.SemaphoreType.DMA,
        pltpu.SemaphoreType.DMA,
    ],
)
def _edge_kernel(table_hbm, gidx_hbm, sidx_hbm, out_hbm,
                 gidx_v, sidx_v, rows_v, acc_sh, sem0, sem1, sem2, sem3):
    c = lax.axis_index("c")
    s = lax.axis_index("s")
    base = s * RPT
    sems = (sem0, sem1, sem2, sem3)

    def _zero_body(i, carry):
        for j in range(D // 16):
            rows_v[0, i, pl.ds(j * 16, 16)] = jnp.zeros((16,), jnp.float32)
        return carry

    lax.fori_loop(0, CHUNK, _zero_body, 0)
    for k in range(RPT // ZROWS):
        pltpu.sync_copy(rows_v.at[0],
                        acc_sh.at[pl.ds(base + k * ZROWS, ZROWS)])
    plsc.subcore_barrier()

    nsteps = IGRP // NBUF

    def _step_body(k, carry):
        for b in range(NBUF):
            j = NBUF * k + b
            pltpu.make_async_copy(table_hbm.at[gidx_v.at[j]],
                                  rows_v.at[b], sems[b]).wait()
            pltpu.sync_copy(rows_v.at[b], acc_sh.at[sidx_v.at[j]], add=True)

            @pl.when(j + NBUF < IGRP)
            def _():
                pltpu.async_copy(table_hbm.at[gidx_v.at[j + NBUF]],
                                 rows_v.at[b], sems[b])
        return carry

    for grp in range(NCH // IGRP):
        pltpu.sync_copy(gidx_hbm.at[c, s, pl.ds(grp * IGRP, IGRP)], gidx_v)
        pltpu.sync_copy(sidx_hbm.at[s, pl.ds(grp * IGRP, IGRP)], sidx_v)
        for b in range(NBUF):
            pltpu.async_copy(table_hbm.at[gidx_v.at[b]],
                             rows_v.at[b], sems[b])
        lax.fori_loop(0, nsteps, _step_body, 0)
    plsc.subcore_barrier()

    pltpu.sync_copy(acc_sh.at[pl.ds(base, RPT)],
                    out_hbm.at[c, pl.ds(base, RPT)])


def _epilogue_body(acc_ref, x_ref, w_ref, b_ref, o_ref):
    sarr = acc_ref[0]
    tarr = acc_ref[1]
    x = x_ref[...]
    agg = tarr / (sarr + 1e-16)
    an = jnp.sqrt(jnp.sum(agg * agg, axis=1, keepdims=True))
    msg = agg / jnp.maximum(an, 1e-12)
    xn = jnp.sqrt(jnp.sum(x * x, axis=1, keepdims=True))
    feats = x + msg * xn
    o_ref[...] = lax.dot_general(
        feats, w_ref[...], (((1,), (1,)), ((), ())),
        preferred_element_type=jnp.float32) + b_ref[...]


_RB = 1000

_epilogue = pl.pallas_call(
    _epilogue_body,
    grid=(N // _RB,),
    in_specs=[
        pl.BlockSpec((NC, _RB, D), lambda i: (0, i, 0)),
        pl.BlockSpec((_RB, D), lambda i: (i, 0)),
        pl.BlockSpec((D, D), lambda i: (0, 0)),
        pl.BlockSpec((1, D), lambda i: (0, 0)),
    ],
    out_specs=pl.BlockSpec((_RB, D), lambda i: (i, 0)),
    out_shape=jax.ShapeDtypeStruct((N, D), jnp.float32),
)


def kernel(node_feats, edge_index, W, b):
    src = edge_index[0]
    dst = edge_index[1]

    padg = jnp.full((EPAD - E,), 2 * N, jnp.int32)
    gidx = jnp.stack([
        jnp.concatenate([src, padg]),
        jnp.concatenate([src + N, padg]),
    ]).reshape(NC, NS, NCH, CHUNK)
    sidx = jnp.concatenate(
        [dst, jnp.zeros((EPAD - E,), jnp.int32)]).reshape(NS, NCH, CHUNK)

    table = _prologue(node_feats)
    acc = _edge_kernel(table, gidx, sidx)
    return _epilogue(acc, node_feats, W, b.reshape(1, D))


# speedup vs baseline: 6.0979x; 1.0117x over previous
import functools

import jax
import jax.numpy as jnp
from jax import lax
from jax.experimental import pallas as pl
from jax.experimental.pallas import tpu as pltpu
from jax.experimental.pallas import tpu_sc as plsc

N = 10000
D = 128
E = 320000
EPS = 1e-07

NC = 2
NS = 16
CHUNK = 64
NBUF = 4
NCH = 320
IGRP = 32
EPT = NCH * CHUNK
EPAD = NS * EPT
NPAD = 10240
RPT = NPAD // NS
ZROWS = 64


def _prologue_body(x_ref, table_ref):
    x = x_ref[...]
    g = jnp.maximum(x, 0.0) + EPS
    gmax = jnp.max(g, axis=0, keepdims=True)
    eg = jnp.exp(g - gmax)
    table_ref[0:N, :] = eg
    table_ref[N:2 * N, :] = g * eg
    table_ref[2 * N:, :] = jnp.zeros((8, D), jnp.float32)


_prologue = pl.pallas_call(
    _prologue_body,
    out_shape=jax.ShapeDtypeStruct((2 * N + 8, D), jnp.float32),
)


_sc_mesh = plsc.VectorSubcoreMesh(core_axis_name="c", subcore_axis_name="s")


@functools.partial(
    pl.kernel,
    mesh=_sc_mesh,
    out_type=jax.ShapeDtypeStruct((NC, NPAD, D), jnp.float32),
    scratch_types=[
        pltpu.VMEM((2, IGRP, CHUNK), jnp.int32),
        pltpu.VMEM((2, IGRP, CHUNK), jnp.int32),
        pltpu.VMEM((NBUF, CHUNK, D), jnp.float32),
        pltpu.VMEM_SHARED((NPAD, D), jnp.float32),
        pltpu.SemaphoreType.DMA,
        pltpu.SemaphoreType.DMA,
        pltpu.SemaphoreType.DMA,
        pltpu.SemaphoreType.DMA,
        pltpu.SemaphoreType.DMA,
    ],
)
def _edge_kernel(table_hbm, gidx_hbm, sidx_hbm, out_hbm,
                 gidx_v, sidx_v, rows_v, acc_sh,
                 sem0, sem1, sem2, sem3, semi):
    c = lax.axis_index("c")
    s = lax.axis_index("s")
    base = s * RPT
    sems = (sem0, sem1, sem2, sem3)

    def _zero_body(i, carry):
        for j in range(D // 16):
            rows_v[0, i, pl.ds(j * 16, 16)] = jnp.zeros((16,), jnp.float32)
        return carry

    lax.fori_loop(0, CHUNK, _zero_body, 0)
    for k in range(RPT // ZROWS):
        pltpu.sync_copy(rows_v.at[0],
                        acc_sh.at[pl.ds(base + k * ZROWS, ZROWS)])
    plsc.subcore_barrier()

    nsteps = IGRP // NBUF
    ngrp = NCH // IGRP

    def _make_step_body(slot):
        def _step_body(k, carry):
            for b in range(NBUF):
                j = NBUF * k + b
                pltpu.make_async_copy(table_hbm.at[gidx_v.at[slot, j]],
                                      rows_v.at[b], sems[b]).wait()
                pltpu.sync_copy(rows_v.at[b],
                                acc_sh.at[sidx_v.at[slot, j]], add=True)

                @pl.when(j + NBUF < IGRP)
                def _():
                    pltpu.async_copy(table_hbm.at[gidx_v.at[slot, j + NBUF]],
                                     rows_v.at[b], sems[b])
            return carry
        return _step_body

    step_bodies = (_make_step_body(0), _make_step_body(1))

    pltpu.sync_copy(gidx_hbm.at[c, s, pl.ds(0, IGRP)], gidx_v.at[0])
    pltpu.sync_copy(sidx_hbm.at[s, pl.ds(0, IGRP)], sidx_v.at[0])
    for grp in range(ngrp):
        slot = grp % 2
        nxt = 1 - slot
        if grp + 1 < ngrp:
            pltpu.async_copy(gidx_hbm.at[c, s, pl.ds((grp + 1) * IGRP, IGRP)],
                             gidx_v.at[nxt], semi)
            pltpu.async_copy(sidx_hbm.at[s, pl.ds((grp + 1) * IGRP, IGRP)],
                             sidx_v.at[nxt], semi)
        for b in range(NBUF):
            pltpu.async_copy(table_hbm.at[gidx_v.at[slot, b]],
                             rows_v.at[b], sems[b])
        lax.fori_loop(0, nsteps, step_bodies[slot], 0)
        if grp + 1 < ngrp:
            pltpu.make_async_copy(
                gidx_hbm.at[c, s, pl.ds((grp + 1) * IGRP, IGRP)],
                gidx_v.at[nxt], semi).wait()
            pltpu.make_async_copy(
                sidx_hbm.at[s, pl.ds((grp + 1) * IGRP, IGRP)],
                sidx_v.at[nxt], semi).wait()
    plsc.subcore_barrier()

    pltpu.sync_copy(acc_sh.at[pl.ds(base, RPT)],
                    out_hbm.at[c, pl.ds(base, RPT)])


def _epilogue_body(acc_ref, x_ref, w_ref, b_ref, o_ref):
    sarr = acc_ref[0]
    tarr = acc_ref[1]
    x = x_ref[...]
    agg = tarr / (sarr + 1e-16)
    an = jnp.sqrt(jnp.sum(agg * agg, axis=1, keepdims=True))
    msg = agg / jnp.maximum(an, 1e-12)
    xn = jnp.sqrt(jnp.sum(x * x, axis=1, keepdims=True))
    feats = x + msg * xn
    o_ref[...] = lax.dot_general(
        feats, w_ref[...], (((1,), (1,)), ((), ())),
        preferred_element_type=jnp.float32) + b_ref[...]


_RB = 1000

_epilogue = pl.pallas_call(
    _epilogue_body,
    grid=(N // _RB,),
    in_specs=[
        pl.BlockSpec((NC, _RB, D), lambda i: (0, i, 0)),
        pl.BlockSpec((_RB, D), lambda i: (i, 0)),
        pl.BlockSpec((D, D), lambda i: (0, 0)),
        pl.BlockSpec((1, D), lambda i: (0, 0)),
    ],
    out_specs=pl.BlockSpec((_RB, D), lambda i: (i, 0)),
    out_shape=jax.ShapeDtypeStruct((N, D), jnp.float32),
)


def kernel(node_feats, edge_index, W, b):
    src = edge_index[0]
    dst = edge_index[1]

    padg = jnp.full((EPAD - E,), 2 * N, jnp.int32)
    gidx = jnp.stack([
        jnp.concatenate([src, padg]),
        jnp.concatenate([src + N, padg]),
    ]).reshape(NC, NS, NCH, CHUNK)
    sidx = jnp.concatenate(
        [dst, jnp.zeros((EPAD - E,), jnp.int32)]).reshape(NS, NCH, CHUNK)

    table = _prologue(node_feats)
    acc = _edge_kernel(table, gidx, sidx)
    return _epilogue(acc, node_feats, W, b.reshape(1, D))
